# baseline (speedup 1.0000x reference)
import functools

import jax
import jax.numpy as jnp
from jax import lax
from jax.experimental import pallas as pl
from jax.experimental.pallas import tpu as pltpu
from jax.experimental.pallas import tpu_sc as plsc

FH, FW, K = 48, 48, 9
N = FH * FW * K
NS = 16
PER = N // NS
VR = PER // 16
NUM_ROIS = 300
IMG = 768.0
NEG = float("-inf")
BIG = 1 << 30
ROIS_PAD = 4 * NUM_ROIS + 16


def _perm(v, idx):
    return lax.gather(
        v, idx[:, None],
        lax.GatherDimensionNumbers(offset_dims=(), collapsed_slice_dims=(0,),
                                   start_index_map=(0,)),
        slice_sizes=(1,),
        mode=lax.GatherScatterMode.PROMISE_IN_BOUNDS)


def _allmax(v, shuf):
    for s in shuf:
        v = jnp.maximum(v, _perm(v, s))
    return v


def _allmin(v, shuf):
    for s in shuf:
        v = jnp.minimum(v, _perm(v, s))
    return v


def _nms_kernel(l0_h, l1_h, ty_h, tx_h, th_h, tw_h,
                ay1_h, ax1_h, ay2_h, ax2_h, out_h,
                l0_v, l1_v, ty_v, tx_v, th_v, tw_v,
                a1_v, a2_v, a3_v, a4_v,
                y1_v, x1_v, y2_v, x2_v, ar_v, sc_v,
                row_v, cand_v, rois_v, shared):
    cid = lax.axis_index("c")
    sid = lax.axis_index("s")
    base = sid * PER
    iota = lax.broadcasted_iota(jnp.int32, (16,), 0)
    shuf = [iota ^ s for s in (8, 4, 2, 1)]

    for src, dst in ((l0_h, l0_v), (l1_h, l1_v), (ty_h, ty_v), (tx_h, tx_v),
                     (th_h, th_v), (tw_h, tw_v), (ay1_h, a1_v), (ax1_h, a2_v),
                     (ay2_h, a3_v), (ax2_h, a4_v)):
        pltpu.sync_copy(src.at[pl.ds(base, PER)], dst)

    def zero_j(j, c):
        rois_v[pl.ds(j * 16, 16)] = jnp.zeros((16,), jnp.float32)
        return c
    lax.fori_loop(0, ROIS_PAD // 16, zero_j, 0)

    bv0 = jnp.full((16,), NEG, jnp.float32)
    bi0 = jnp.zeros((16,), jnp.int32)

    def decode_j(j, carry):
        bv, bi = carry
        sl = pl.ds(j * 16, 16)
        l0 = l0_v[sl]
        l1 = l1_v[sl]
        m = jnp.maximum(l0, l1)
        e0 = jnp.exp(l0 - m)
        e1 = jnp.exp(l1 - m)
        p = e1 / (e0 + e1)
        ay1 = a1_v[sl]
        ax1 = a2_v[sl]
        ay2 = a3_v[sl]
        ax2 = a4_v[sl]
        ah = ay2 - ay1
        aw = ax2 - ax1
        cy = ay1 + ah * 0.5 + ty_v[sl] * ah
        cx = ax1 + aw * 0.5 + tx_v[sl] * aw
        bh = ah * jnp.exp(th_v[sl])
        bw = aw * jnp.exp(tw_v[sl])
        y1 = jnp.minimum(jnp.maximum(cy - bh * 0.5, 0.0), IMG)
        x1 = jnp.minimum(jnp.maximum(cx - bw * 0.5, 0.0), IMG)
        y2 = jnp.minimum(jnp.maximum(cy + bh * 0.5, 0.0), IMG)
        x2 = jnp.minimum(jnp.maximum(cx + bw * 0.5, 0.0), IMG)
        area = jnp.maximum(y2 - y1, 0.0) * jnp.maximum(x2 - x1, 0.0)
        s = jnp.where(p >= 0.5, p, NEG)
        y1_v[sl] = y1
        x1_v[sl] = x1
        y2_v[sl] = y2
        x2_v[sl] = x2
        ar_v[sl] = area
        sc_v[sl] = s
        li = j * 16 + iota
        gt = s > bv
        return jnp.where(gt, s, bv), jnp.where(gt, li, bi)

    bv, bi = lax.fori_loop(0, VR, decode_j, (bv0, bi0))

    def publish_reduce(bv, bi, par):
        tb = par * 256
        bm = _allmax(bv, shuf)
        bloc = _allmin(jnp.where(bv == bm, bi, BIG), shuf)
        bsafe = jnp.minimum(bloc, PER - 1)
        wy1 = plsc.load_gather(y1_v, [bsafe])
        wx1 = plsc.load_gather(x1_v, [bsafe])
        wy2 = plsc.load_gather(y2_v, [bsafe])
        wx2 = plsc.load_gather(x2_v, [bsafe])
        wa = plsc.load_gather(ar_v, [bsafe])
        gidxf = (bsafe + base).astype(jnp.float32)
        row = jnp.where(iota == 0, bm,
              jnp.where(iota == 1, gidxf,
              jnp.where(iota == 2, wy1,
              jnp.where(iota == 3, wx1,
              jnp.where(iota == 4, wy2,
              jnp.where(iota == 5, wx2,
              jnp.where(iota == 6, wa, 0.0)))))))
        row_v[...] = row
        pltpu.sync_copy(row_v, shared.at[pl.ds(tb + sid * 16, 16)])
        plsc.subcore_barrier()
        pltpu.sync_copy(shared.at[pl.ds(tb, 256)], cand_v)
        tbase = iota * 16
        sc_c = plsc.load_gather(cand_v, [tbase])
        ix_c = plsc.load_gather(cand_v, [tbase + 1]).astype(jnp.int32)
        gm = _allmax(sc_c, shuf)
        gi = _allmin(jnp.where(sc_c == gm, ix_c, BIG), shuf)
        rb = jnp.minimum(gi // PER, NS - 1) * 16
        gy1 = plsc.load_gather(cand_v, [rb + 2])
        gx1 = plsc.load_gather(cand_v, [rb + 3])
        gy2 = plsc.load_gather(cand_v, [rb + 4])
        gx2 = plsc.load_gather(cand_v, [rb + 5])
        ga = plsc.load_gather(cand_v, [rb + 6])
        return gm, gi, gy1, gx1, gy2, gx2, ga

    st0 = publish_reduce(bv, bi, jnp.int32(0))

    def step(t, st):
        gm, gi, gy1, gx1, gy2, gx2, ga = st
        valid = gm > NEG

        @pl.when((cid == 0) & (sid == 0))
        def _store_roi():
            roi = jnp.where(iota == 0, gy1,
                  jnp.where(iota == 1, gx1,
                  jnp.where(iota == 2, gy2,
                  jnp.where(iota == 3, gx2, 0.0))))
            rois_v[pl.ds(t * 4, 16)] = jnp.where(valid, roi, 0.0)

        gl = gi - base

        def supp_j(j, carry):
            bv, bi = carry
            sl = pl.ds(j * 16, 16)
            s = sc_v[sl]
            yy1 = jnp.maximum(gy1, y1_v[sl])
            xx1 = jnp.maximum(gx1, x1_v[sl])
            yy2 = jnp.minimum(gy2, y2_v[sl])
            xx2 = jnp.minimum(gx2, x2_v[sl])
            ih = jnp.maximum(yy2 - yy1, 0.0)
            iw = jnp.maximum(xx2 - xx1, 0.0)
            inter = ih * iw
            union = ga + ar_v[sl] - inter + 1e-8
            li = j * 16 + iota
            sup = (inter > 0.5 * union) | (li == gl)
            ns = jnp.where(sup, NEG, s)
            sc_v[sl] = ns
            gt = ns > bv
            return jnp.where(gt, ns, bv), jnp.where(gt, li, bi)

        bv, bi = lax.fori_loop(0, VR, supp_j, (bv0, bi0))
        return publish_reduce(bv, bi, (t + 1) % 2)

    lax.fori_loop(0, NUM_ROIS, step, st0)

    @pl.when((cid == 0) & (sid == 0))
    def _write_out():
        pltpu.sync_copy(rois_v.at[pl.ds(0, 4 * NUM_ROIS)], out_h)


@functools.partial(
    pl.kernel,
    out_type=jax.ShapeDtypeStruct((4 * NUM_ROIS,), jnp.float32),
    mesh=plsc.VectorSubcoreMesh(core_axis_name="c", subcore_axis_name="s",
                                num_cores=1, num_subcores=16),
    compiler_params=pltpu.CompilerParams(needs_layout_passes=False),
    scratch_types=[pltpu.VMEM((PER,), jnp.float32) for _ in range(16)]
                  + [pltpu.VMEM((16,), jnp.float32),
                     pltpu.VMEM((256,), jnp.float32),
                     pltpu.VMEM((ROIS_PAD,), jnp.float32),
                     pltpu.VMEM_SHARED((512,), jnp.float32)],
)
def _nms_call(*args):
    _nms_kernel(*args)


def kernel(x, anchors):
    t = x.reshape(N, 6)
    a = anchors.reshape(N, 4)
    cols = tuple(t[:, i] for i in range(6))
    acols = tuple(a[:, i] for i in range(4))
    rois = _nms_call(*cols, *acols)
    return rois.reshape(1, NUM_ROIS, 4)


# speedup vs baseline: 1.7984x; 1.7984x over previous
import functools

import jax
import jax.numpy as jnp
from jax import lax
from jax.experimental import pallas as pl
from jax.experimental.pallas import tpu as pltpu
from jax.experimental.pallas import tpu_sc as plsc

FH, FW, K = 48, 48, 9
N = FH * FW * K
NS = 16
PER = N // NS
VR = PER // 16
NUM_ROIS = 300
IMG = 768.0
NEG = float("-inf")
BIG = 1 << 30
ROIS_PAD = 4 * NUM_ROIS + 16


def _perm(v, idx):
    return lax.gather(
        v, idx[:, None],
        lax.GatherDimensionNumbers(offset_dims=(), collapsed_slice_dims=(0,),
                                   start_index_map=(0,)),
        slice_sizes=(1,),
        mode=lax.GatherScatterMode.PROMISE_IN_BOUNDS)


def _allmax(v, shuf):
    for s in shuf:
        v = jnp.maximum(v, _perm(v, s))
    return v


def _allmin(v, shuf):
    for s in shuf:
        v = jnp.minimum(v, _perm(v, s))
    return v


def _nms_kernel(l0_h, l1_h, ty_h, tx_h, th_h, tw_h,
                ay1_h, ax1_h, ay2_h, ax2_h, out_h,
                l0_v, l1_v, ty_v, tx_v, th_v, tw_v,
                a1_v, a2_v, a3_v, a4_v,
                y1_v, x1_v, y2_v, x2_v, ar_v, sc_v,
                row_v, cand_v, rois_v, shared):
    cid = lax.axis_index("c")
    sid = lax.axis_index("s")
    base = sid * PER
    iota = lax.broadcasted_iota(jnp.int32, (16,), 0)
    shuf = [iota ^ s for s in (8, 4, 2, 1)]

    for src, dst in ((l0_h, l0_v), (l1_h, l1_v), (ty_h, ty_v), (tx_h, tx_v),
                     (th_h, th_v), (tw_h, tw_v), (ay1_h, a1_v), (ax1_h, a2_v),
                     (ay2_h, a3_v), (ax2_h, a4_v)):
        pltpu.sync_copy(src.at[pl.ds(base, PER)], dst)

    def zero_j(j, c):
        rois_v[pl.ds(j * 16, 16)] = jnp.zeros((16,), jnp.float32)
        return c
    lax.fori_loop(0, ROIS_PAD // 16, zero_j, 0)

    bv0 = jnp.full((16,), NEG, jnp.float32)
    bi0 = jnp.zeros((16,), jnp.int32)

    def decode_j(j, carry):
        bv, bi = carry
        sl = pl.ds(j * 16, 16)
        l0 = l0_v[sl]
        l1 = l1_v[sl]
        m = jnp.maximum(l0, l1)
        e0 = jnp.exp(l0 - m)
        e1 = jnp.exp(l1 - m)
        p = e1 / (e0 + e1)
        ay1 = a1_v[sl]
        ax1 = a2_v[sl]
        ay2 = a3_v[sl]
        ax2 = a4_v[sl]
        ah = ay2 - ay1
        aw = ax2 - ax1
        cy = ay1 + ah * 0.5 + ty_v[sl] * ah
        cx = ax1 + aw * 0.5 + tx_v[sl] * aw
        bh = ah * jnp.exp(th_v[sl])
        bw = aw * jnp.exp(tw_v[sl])
        y1 = jnp.minimum(jnp.maximum(cy - bh * 0.5, 0.0), IMG)
        x1 = jnp.minimum(jnp.maximum(cx - bw * 0.5, 0.0), IMG)
        y2 = jnp.minimum(jnp.maximum(cy + bh * 0.5, 0.0), IMG)
        x2 = jnp.minimum(jnp.maximum(cx + bw * 0.5, 0.0), IMG)
        area = jnp.maximum(y2 - y1, 0.0) * jnp.maximum(x2 - x1, 0.0)
        s = jnp.where(p >= 0.5, p, NEG)
        y1_v[sl] = y1
        x1_v[sl] = x1
        y2_v[sl] = y2
        x2_v[sl] = x2
        ar_v[sl] = area
        sc_v[sl] = s
        li = j * 16 + iota
        gt = s > bv
        return jnp.where(gt, s, bv), jnp.where(gt, li, bi)

    bv, bi = lax.fori_loop(0, VR, decode_j, (bv0, bi0))

    def publish_reduce(bv, bi):
        bm = _allmax(bv, shuf)
        bloc = _allmin(jnp.where(bv == bm, bi, BIG), shuf)
        wy1 = plsc.load_gather(y1_v, [bloc])
        wx1 = plsc.load_gather(x1_v, [bloc])
        wy2 = plsc.load_gather(y2_v, [bloc])
        wx2 = plsc.load_gather(x2_v, [bloc])
        wa = plsc.load_gather(ar_v, [bloc])
        gidxf = (bloc + base).astype(jnp.float32)
        row = jnp.where(iota == 0, bm,
              jnp.where(iota == 1, gidxf,
              jnp.where(iota == 2, wy1,
              jnp.where(iota == 3, wx1,
              jnp.where(iota == 4, wy2,
              jnp.where(iota == 5, wx2,
              jnp.where(iota == 6, wa, 0.0)))))))
        row_v[...] = row
        pltpu.sync_copy(row_v, shared.at[pl.ds(sid * 16, 16)])
        plsc.subcore_barrier()
        pltpu.sync_copy(shared, cand_v)
        tbase = iota * 16
        sc_c = plsc.load_gather(cand_v, [tbase])
        ix_c = plsc.load_gather(cand_v, [tbase + 1]).astype(jnp.int32)
        gm = _allmax(sc_c, shuf)
        gi = _allmin(jnp.where(sc_c == gm, ix_c, BIG), shuf)
        rb = (gi // PER) * 16
        gy1 = plsc.load_gather(cand_v, [rb + 2])
        gx1 = plsc.load_gather(cand_v, [rb + 3])
        gy2 = plsc.load_gather(cand_v, [rb + 4])
        gx2 = plsc.load_gather(cand_v, [rb + 5])
        ga = plsc.load_gather(cand_v, [rb + 6])
        return gm, gi, gy1, gx1, gy2, gx2, ga

    st0 = publish_reduce(bv, bi)

    def step(t, st):
        gm, gi, gy1, gx1, gy2, gx2, ga = st
        valid = gm > NEG

        @pl.when((cid == 0) & (sid == 0))
        def _store_roi():
            roi = jnp.where(iota == 0, gy1,
                  jnp.where(iota == 1, gx1,
                  jnp.where(iota == 2, gy2,
                  jnp.where(iota == 3, gx2, 0.0))))
            rois_v[pl.ds(t * 4, 16)] = jnp.where(valid, roi, 0.0)

        plsc.subcore_barrier()

        gl = gi - base

        def supp_j(j, carry):
            bv, bi, li = carry
            sl = pl.ds(j * 16, 16)
            s = sc_v[sl]
            yy1 = jnp.maximum(gy1, y1_v[sl])
            xx1 = jnp.maximum(gx1, x1_v[sl])
            yy2 = jnp.minimum(gy2, y2_v[sl])
            xx2 = jnp.minimum(gx2, x2_v[sl])
            ih = jnp.maximum(yy2 - yy1, 0.0)
            iw = jnp.maximum(xx2 - xx1, 0.0)
            inter = ih * iw
            union = ga + ar_v[sl] - inter + 1e-8
            sup = (inter > 0.5 * union) | (li == gl)
            ns = jnp.where(sup, NEG, s)
            sc_v[sl] = ns
            gt = ns > bv
            return jnp.where(gt, ns, bv), jnp.where(gt, li, bi), li + 16

        bv, bi, _ = lax.fori_loop(0, VR, supp_j, (bv0, bi0, iota))
        return publish_reduce(bv, bi)

    lax.fori_loop(0, NUM_ROIS, step, st0)

    @pl.when((cid == 0) & (sid == 0))
    def _write_out():
        pltpu.sync_copy(rois_v.at[pl.ds(0, 4 * NUM_ROIS)], out_h)


@functools.partial(
    pl.kernel,
    out_type=jax.ShapeDtypeStruct((4 * NUM_ROIS,), jnp.float32),
    mesh=plsc.VectorSubcoreMesh(core_axis_name="c", subcore_axis_name="s",
                                num_cores=1, num_subcores=16),
    compiler_params=pltpu.CompilerParams(needs_layout_passes=False),
    scratch_types=[pltpu.VMEM((PER,), jnp.float32) for _ in range(16)]
                  + [pltpu.VMEM((16,), jnp.float32),
                     pltpu.VMEM((256,), jnp.float32),
                     pltpu.VMEM((ROIS_PAD,), jnp.float32),
                     pltpu.VMEM_SHARED((256,), jnp.float32)],
)
def _nms_call(*args):
    _nms_kernel(*args)


def kernel(x, anchors):
    t = x.reshape(N, 6)
    a = anchors.reshape(N, 4)
    cols = tuple(t[:, i] for i in range(6))
    acols = tuple(a[:, i] for i in range(4))
    rois = _nms_call(*cols, *acols)
    return rois.reshape(1, NUM_ROIS, 4)


# speedup vs baseline: 2.0267x; 1.1269x over previous
import functools

import jax
import jax.numpy as jnp
from jax import lax
from jax.experimental import pallas as pl
from jax.experimental.pallas import tpu as pltpu
from jax.experimental.pallas import tpu_sc as plsc

FH, FW, K = 48, 48, 9
N = FH * FW * K
NS = 16
PER = N // NS
VR = PER // 16
NUM_ROIS = 300
IMG = 768.0
NEG = float("-inf")
BIG = 1 << 30
ROIS_PAD = 4 * NUM_ROIS + 32


def _perm(v, idx):
    return lax.gather(
        v, idx[:, None],
        lax.GatherDimensionNumbers(offset_dims=(), collapsed_slice_dims=(0,),
                                   start_index_map=(0,)),
        slice_sizes=(1,),
        mode=lax.GatherScatterMode.PROMISE_IN_BOUNDS)


def _allmax(v, shuf):
    for s in shuf:
        v = jnp.maximum(v, _perm(v, s))
    return v


def _allmin(v, shuf):
    for s in shuf:
        v = jnp.minimum(v, _perm(v, s))
    return v


def _nms_kernel(l0_h, l1_h, ty_h, tx_h, th_h, tw_h,
                ay1_h, ax1_h, ay2_h, ax2_h, out_h,
                l0_v, l1_v, ty_v, tx_v, th_v, tw_v,
                a1_v, a2_v, a3_v, a4_v,
                y1_v, x1_v, y2_v, x2_v, ar_v, sc_v,
                row_v, cand_v, flag_v, rois_v, shared):
    cid = lax.axis_index("c")
    sid = lax.axis_index("s")
    base = sid * PER
    iota = lax.broadcasted_iota(jnp.int32, (16,), 0)
    shuf = [iota ^ s for s in (8, 4, 2, 1)]

    for src, dst in ((l0_h, l0_v), (l1_h, l1_v), (ty_h, ty_v), (tx_h, tx_v),
                     (th_h, th_v), (tw_h, tw_v), (ay1_h, a1_v), (ax1_h, a2_v),
                     (ay2_h, a3_v), (ax2_h, a4_v)):
        pltpu.sync_copy(src.at[pl.ds(base, PER)], dst)

    def zero_j(j, c):
        rois_v[pl.ds(j * 16, 16)] = jnp.zeros((16,), jnp.float32)
        return c
    lax.fori_loop(0, ROIS_PAD // 16, zero_j, 0)

    bv0 = jnp.full((16,), NEG, jnp.float32)
    bi0 = jnp.zeros((16,), jnp.int32)

    def top2_update(s, li, bv1, bi1, bv2, bi2):
        gt1 = s > bv1
        gt2 = s > bv2
        nbv2 = jnp.where(gt1, bv1, jnp.where(gt2, s, bv2))
        nbi2 = jnp.where(gt1, bi1, jnp.where(gt2, li, bi2))
        nbv1 = jnp.where(gt1, s, bv1)
        nbi1 = jnp.where(gt1, li, bi1)
        return nbv1, nbi1, nbv2, nbi2

    def decode_j(j, carry):
        bv1, bi1, bv2, bi2, li = carry
        sl = pl.ds(j * 16, 16)
        l0 = l0_v[sl]
        l1 = l1_v[sl]
        m = jnp.maximum(l0, l1)
        e0 = jnp.exp(l0 - m)
        e1 = jnp.exp(l1 - m)
        p = e1 / (e0 + e1)
        ay1 = a1_v[sl]
        ax1 = a2_v[sl]
        ay2 = a3_v[sl]
        ax2 = a4_v[sl]
        ah = ay2 - ay1
        aw = ax2 - ax1
        cy = ay1 + ah * 0.5 + ty_v[sl] * ah
        cx = ax1 + aw * 0.5 + tx_v[sl] * aw
        bh = ah * jnp.exp(th_v[sl])
        bw = aw * jnp.exp(tw_v[sl])
        y1 = jnp.minimum(jnp.maximum(cy - bh * 0.5, 0.0), IMG)
        x1 = jnp.minimum(jnp.maximum(cx - bw * 0.5, 0.0), IMG)
        y2 = jnp.minimum(jnp.maximum(cy + bh * 0.5, 0.0), IMG)
        x2 = jnp.minimum(jnp.maximum(cx + bw * 0.5, 0.0), IMG)
        area = jnp.maximum(y2 - y1, 0.0) * jnp.maximum(x2 - x1, 0.0)
        s = jnp.where(p >= 0.5, p, NEG)
        y1_v[sl] = y1
        x1_v[sl] = x1
        y2_v[sl] = y2
        x2_v[sl] = x2
        ar_v[sl] = area
        sc_v[sl] = s
        bv1, bi1, bv2, bi2 = top2_update(s, li, bv1, bi1, bv2, bi2)
        return bv1, bi1, bv2, bi2, li + 16

    bv1, bi1, bv2, bi2, _ = lax.fori_loop(
        0, VR, decode_j, (bv0, bi0, bv0, bi0, iota))

    def publish_reduce(bv1, bi1, bv2, bi2):
        t1 = _allmax(bv1, shuf)
        i1 = _allmin(jnp.where(bv1 == t1, bi1, BIG), shuf)
        win = bi1 == i1
        sec_v = jnp.where(win, bv2, bv1)
        sec_i = jnp.where(win, bi2, bi1)
        t2 = _allmax(sec_v, shuf)
        i2 = _allmin(jnp.where(sec_v == t2, sec_i, BIG), shuf)
        i1c = jnp.minimum(i1, PER - 1)
        i2c = jnp.minimum(i2, PER - 1)
        wy1 = plsc.load_gather(y1_v, [i1c])
        wx1 = plsc.load_gather(x1_v, [i1c])
        wy2 = plsc.load_gather(y2_v, [i1c])
        wx2 = plsc.load_gather(x2_v, [i1c])
        wa = plsc.load_gather(ar_v, [i1c])
        vy1 = plsc.load_gather(y1_v, [i2c])
        vx1 = plsc.load_gather(x1_v, [i2c])
        vy2 = plsc.load_gather(y2_v, [i2c])
        vx2 = plsc.load_gather(x2_v, [i2c])
        va = plsc.load_gather(ar_v, [i2c])
        gidx1 = (i1c + base).astype(jnp.float32)
        gidx2 = (i2c + base).astype(jnp.float32)
        row = jnp.where(iota == 0, t1,
              jnp.where(iota == 1, gidx1,
              jnp.where(iota == 2, wy1,
              jnp.where(iota == 3, wx1,
              jnp.where(iota == 4, wy2,
              jnp.where(iota == 5, wx2,
              jnp.where(iota == 6, wa,
              jnp.where(iota == 7, t2,
              jnp.where(iota == 8, gidx2,
              jnp.where(iota == 9, vy1,
              jnp.where(iota == 10, vx1,
              jnp.where(iota == 11, vy2,
              jnp.where(iota == 12, vx2,
              jnp.where(iota == 13, va, 0.0))))))))))))))
        row_v[...] = row
        pltpu.sync_copy(row_v, shared.at[pl.ds(sid * 16, 16)])
        plsc.subcore_barrier()
        pltpu.sync_copy(shared, cand_v)
        tbase = iota * 16
        sc1 = plsc.load_gather(cand_v, [tbase])
        ix1 = plsc.load_gather(cand_v, [tbase + 1]).astype(jnp.int32)
        gm1 = _allmax(sc1, shuf)
        gi1 = _allmin(jnp.where(sc1 == gm1, ix1, BIG), shuf)
        rb1 = (gi1 // PER) * 16
        gy1 = plsc.load_gather(cand_v, [rb1 + 2])
        gx1 = plsc.load_gather(cand_v, [rb1 + 3])
        gy2 = plsc.load_gather(cand_v, [rb1 + 4])
        gx2 = plsc.load_gather(cand_v, [rb1 + 5])
        ga = plsc.load_gather(cand_v, [rb1 + 6])
        winr = ix1 == gi1
        sc2c = jnp.where(winr, plsc.load_gather(cand_v, [tbase + 7]), sc1)
        ix2c = jnp.where(winr,
                         plsc.load_gather(cand_v, [tbase + 8]).astype(jnp.int32),
                         ix1)
        gm2 = _allmax(sc2c, shuf)
        gi2 = _allmin(jnp.where(sc2c == gm2, ix2c, BIG), shuf)
        gi2c = jnp.minimum(gi2, N - 1)
        rb2 = (gi2c // PER) * 16
        slot1_ix = plsc.load_gather(cand_v, [rb2 + 1]).astype(jnp.int32)
        off2 = jnp.where(slot1_ix == gi2c, rb2 + 2, rb2 + 9)
        hy1 = plsc.load_gather(cand_v, [off2])
        hx1 = plsc.load_gather(cand_v, [off2 + 1])
        hy2 = plsc.load_gather(cand_v, [off2 + 2])
        hx2 = plsc.load_gather(cand_v, [off2 + 3])
        ha = plsc.load_gather(cand_v, [off2 + 4])
        yy1 = jnp.maximum(gy1, hy1)
        xx1 = jnp.maximum(gx1, hx1)
        yy2 = jnp.minimum(gy2, hy2)
        xx2 = jnp.minimum(gx2, hx2)
        inter = jnp.maximum(yy2 - yy1, 0.0) * jnp.maximum(xx2 - xx1, 0.0)
        union = ga + ha - inter + 1e-8
        c2vec = (gm2 > NEG) & jnp.logical_not(inter > 0.5 * union)
        c2s = c2vec.astype(jnp.int32)[0]
        z = c2vec.astype(jnp.float32)
        hy1 = hy1 * z
        hx1 = hx1 * z
        hy2 = hy2 * z
        hx2 = hx2 * z
        ha = ha * z
        gi2m = jnp.where(c2vec, gi2, -BIG)
        return (c2s, gm1, gi1, gy1, gx1, gy2, gx2, ga,
                gi2m, hy1, hx1, hy2, hx2, ha)

    st0 = publish_reduce(bv1, bi1, bv2, bi2)

    def cond(carry):
        return carry[0] < NUM_ROIS

    def body(carry):
        (t, c2s, gm1, gi1, gy1, gx1, gy2, gx2, ga,
         gi2m, hy1, hx1, hy2, hx2, ha) = carry
        valid = gm1 > NEG

        @pl.when((cid == 0) & (sid == 0))
        def _store_roi1():
            roi = jnp.where(iota == 0, gy1,
                  jnp.where(iota == 1, gx1,
                  jnp.where(iota == 2, gy2,
                  jnp.where(iota == 3, gx2, 0.0))))
            rois_v[pl.ds(t * 4, 16)] = jnp.where(valid, roi, 0.0)

        @pl.when((cid == 0) & (sid == 0) & (c2s == 1))
        def _store_roi2():
            roi = jnp.where(iota == 0, hy1,
                  jnp.where(iota == 1, hx1,
                  jnp.where(iota == 2, hy2,
                  jnp.where(iota == 3, hx2, 0.0))))
            rois_v[pl.ds(t * 4 + 4, 16)] = roi

        plsc.subcore_barrier()

        gl1 = gi1 - base
        gl2 = gi2m - base

        def supp_j(j, carry):
            bv1, bi1, bv2, bi2, li = carry
            sl = pl.ds(j * 16, 16)
            s = sc_v[sl]
            py1 = y1_v[sl]
            px1 = x1_v[sl]
            py2 = y2_v[sl]
            px2 = x2_v[sl]
            pa = ar_v[sl]
            yy1 = jnp.maximum(gy1, py1)
            xx1 = jnp.maximum(gx1, px1)
            yy2 = jnp.minimum(gy2, py2)
            xx2 = jnp.minimum(gx2, px2)
            ih = jnp.maximum(yy2 - yy1, 0.0)
            iw = jnp.maximum(xx2 - xx1, 0.0)
            inter = ih * iw
            union = ga + pa - inter + 1e-8
            sup = (inter > 0.5 * union) | (li == gl1)
            by1 = jnp.maximum(hy1, py1)
            bx1 = jnp.maximum(hx1, px1)
            by2 = jnp.minimum(hy2, py2)
            bx2 = jnp.minimum(hx2, px2)
            bh_ = jnp.maximum(by2 - by1, 0.0)
            bw_ = jnp.maximum(bx2 - bx1, 0.0)
            binter = bh_ * bw_
            bunion = ha + pa - binter + 1e-8
            sup = sup | (binter > 0.5 * bunion) | (li == gl2)
            ns = jnp.where(sup, NEG, s)
            sc_v[sl] = ns
            bv1, bi1, bv2, bi2 = top2_update(ns, li, bv1, bi1, bv2, bi2)
            return bv1, bi1, bv2, bi2, li + 16

        bv1, bi1, bv2, bi2, _ = lax.fori_loop(
            0, VR, supp_j, (bv0, bi0, bv0, bi0, iota))
        nxt = publish_reduce(bv1, bi1, bv2, bi2)
        return (t + 1 + c2s,) + nxt

    lax.while_loop(cond, body, (jnp.int32(0),) + st0)

    @pl.when((cid == 0) & (sid == 0))
    def _write_out():
        pltpu.sync_copy(rois_v.at[pl.ds(0, 4 * NUM_ROIS)], out_h)


@functools.partial(
    pl.kernel,
    out_type=jax.ShapeDtypeStruct((4 * NUM_ROIS,), jnp.float32),
    mesh=plsc.VectorSubcoreMesh(core_axis_name="c", subcore_axis_name="s",
                                num_cores=1, num_subcores=16),
    compiler_params=pltpu.CompilerParams(needs_layout_passes=False),
    scratch_types=[pltpu.VMEM((PER,), jnp.float32) for _ in range(16)]
                  + [pltpu.VMEM((16,), jnp.float32),
                     pltpu.VMEM((256,), jnp.float32),
                     pltpu.VMEM((16,), jnp.int32),
                     pltpu.VMEM((ROIS_PAD,), jnp.float32),
                     pltpu.VMEM_SHARED((256,), jnp.float32)],
)
def _nms_call(*args):
    _nms_kernel(*args)


def kernel(x, anchors):
    t = x.reshape(N, 6)
    a = anchors.reshape(N, 4)
    cols = tuple(t[:, i] for i in range(6))
    acols = tuple(a[:, i] for i in range(4))
    rois = _nms_call(*cols, *acols)
    return rois.reshape(1, NUM_ROIS, 4)
